# baseline (speedup 1.0000x reference)
import functools

import jax
import jax.numpy as jnp
from jax import lax
from jax.experimental import pallas as pl
from jax.experimental.pallas import tpu as pltpu
from jax.experimental.pallas import tpu_sc as plsc

_F32 = jnp.float32
_I32 = jnp.int32

_ROW_OFF = (0, 10, 12, 16, 20)
_COL_OFF = (0, 25, 50, 75, 100, 128)
_BIAS_ROW = 28
_N_E_ROWS = 32
_N_OUT = 40
_IDX_PAD = 64
_GROUP = 8
_ROWS_PER_WORKER = 2
_N_WORKERS = _N_OUT // _ROWS_PER_WORKER


def _project_body(lvl_ref, typ_ref, fea_ref, exc_ref, par_ref, w_ref, b_ref,
                  i0_ref, i1_ref, i2_ref, i3_ref, i4_ref, e_ref, idxc_ref):
    w = w_ref[...]
    dn = (((1,), (1,)), ((), ()))
    blocks = []
    for t_ref, k in zip((lvl_ref, typ_ref, fea_ref, exc_ref, par_ref),
                        range(5)):
        w_slice = w[:, _COL_OFF[k]:_COL_OFF[k + 1]]
        blocks.append(lax.dot_general(t_ref[...], w_slice, dn,
                                      preferred_element_type=_F32))
    blocks.append(jnp.reshape(b_ref[...], (1, 128)))
    blocks.append(jnp.zeros((3, 128), _F32))
    e_ref[...] = jnp.concatenate(blocks, axis=0)

    pad = jnp.zeros((1, _IDX_PAD - _N_OUT), _I32)
    rows = [
        jnp.concatenate([jnp.reshape(i_ref[...] + off, (1, _N_OUT)), pad],
                        axis=1)
        for i_ref, off in zip((i0_ref, i1_ref, i2_ref, i3_ref, i4_ref),
                              _ROW_OFF)
    ]
    idxc_ref[...] = jnp.concatenate(rows, axis=0)


def _project(lvl, typ, fea, exc, par, w, b, i0, i1, i2, i3, i4):
    return pl.pallas_call(
        _project_body,
        out_shape=[
            jax.ShapeDtypeStruct((_N_E_ROWS, 128), _F32),
            jax.ShapeDtypeStruct((5, _IDX_PAD), _I32),
        ],
    )(lvl, typ, fea, exc, par, w, b, i0, i1, i2, i3, i4)


def _lane_pick(vec, idx):
    dn = lax.GatherDimensionNumbers(offset_dims=(), collapsed_slice_dims=(0,),
                                    start_index_map=(0,))
    return lax.gather(vec, idx[:, None], dn, slice_sizes=(1,),
                      mode=lax.GatherScatterMode.PROMISE_IN_BOUNDS)


def _assemble16(idxc_v, first_row, h, pos):
    base = pl.ds(first_row, 16)
    val = _lane_pick(idxc_v[0, base], h)
    for p in range(1, 5):
        val = jnp.where(pos == p, _lane_pick(idxc_v[p, base], h), val)
    return jnp.where(pos >= 5, pos + (_BIAS_ROW - 5), val)


def _sum_groups(rows_v, out_v, n_rows):
    for j in range(128 // 16):
        sl = pl.ds(j * 16, 16)
        for g in range(n_rows):
            acc = rows_v[g * _GROUP, sl]
            for i in range(1, _GROUP):
                acc = acc + rows_v[g * _GROUP + i, sl]
            out_v[g, sl] = acc


def _gather_sum_body(e_hbm, idxc_hbm, out_hbm, idxc_v, idx_v, rows_v, out_v,
                     sem):
    wid = lax.axis_index("s")

    pltpu.sync_copy(idxc_hbm, idxc_v)
    lanes = lax.iota(_I32, 16)
    h = lanes >> 3
    pos = lanes & 7

    @pl.when(wid < 8)
    def _():
        r0 = 4 * wid
        idx_v[pl.ds(0, 16)] = _assemble16(idxc_v, r0, h, pos)
        idx_v[pl.ds(16, 16)] = _assemble16(idxc_v, r0, h + 2, pos)
        pltpu.async_copy(e_hbm.at[idx_v], rows_v, sem).wait()
        _sum_groups(rows_v, out_v, 4)
        pltpu.sync_copy(out_v, out_hbm.at[pl.ds(r0, 4)])

    @pl.when((wid >= 8) & (wid < 12))
    def _():
        r0 = 32 + 2 * (wid - 8)
        idx_v[pl.ds(0, 16)] = _assemble16(idxc_v, r0, h, pos)
        pltpu.async_copy(e_hbm.at[idx_v.at[pl.ds(0, 16)]],
                         rows_v.at[pl.ds(0, 16)], sem).wait()
        _sum_groups(rows_v, out_v, 2)
        pltpu.sync_copy(out_v.at[pl.ds(0, 2)], out_hbm.at[pl.ds(r0, 2)])


@functools.lru_cache(maxsize=1)
def _make_gather_sum():
    return pl.kernel(
        _gather_sum_body,
        out_type=jax.ShapeDtypeStruct((_N_OUT, 128), _F32),
        mesh=plsc.VectorSubcoreMesh(core_axis_name="c", subcore_axis_name="s",
                                    num_cores=1),
        scratch_types=[
            pltpu.VMEM((5, _IDX_PAD), _I32),
            pltpu.VMEM((32,), _I32),
            pltpu.VMEM((32, 128), _F32),
            pltpu.VMEM((4, 128), _F32),
            pltpu.SemaphoreType.DMA,
        ],
    )


def kernel(level_idx, type_idx, feature_idx, exchange_idx, pair_idx,
           level_table, type_table, feature_table, exchange_table, pair_table,
           W, b):
    e, idxc = _project(level_table, type_table, feature_table, exchange_table,
                       pair_table, W, b, level_idx, type_idx, feature_idx,
                       exchange_idx, pair_idx)
    return _make_gather_sum()(e, idxc)
